# Initial kernel scaffold; baseline (speedup 1.0000x reference)
import jax
import jax.numpy as jnp
from jax.experimental import pallas as pl


def kernel(positions, t, atom_type_ids, params):
    raise NotImplementedError("write your pallas kernel here")


# speedup vs baseline: 73.8476x; 73.8476x over previous
import math

import jax
import jax.numpy as jnp
from jax.experimental import pallas as pl
from jax.experimental.pallas import tpu as pltpu

_B = 4
_N = 128
_H = 128
_NL = 5
_NRBF = 20
_CUTOFF = 10.0
_RBF_W = _CUTOFF / (_NRBF - 1)
_INV_SQRT_NB = 1.0 / math.sqrt(_N - 1)
_RC = 32
_NCHUNK = (_B * _N) // _RC


def _silu(x):
    return x * jax.nn.sigmoid(x)


def _ln(x, g, b):
    m = jnp.mean(x, axis=-1, keepdims=True)
    var = jnp.mean((x - m) ** 2, axis=-1, keepdims=True)
    return (x - m) / jnp.sqrt(var + 1e-5) * g + b


def _mm(a, b):
    return jax.lax.dot_general(a, b, (((1,), (0,)), ((), ())),
                               preferred_element_type=jnp.float32)


def _painn_body(posc, posr, temb, onehot, atom_emb, atype_emb,
                tproj_wt, tproj_b,
                iln_g, iln_b, ictx1_wt, ictx1_b, ictx2_wt, ictx2_b,
                filt1_wt, filt1_b, filt2_wt, filt2_b,
                mln_g, mln_b, u_wt, v_wt,
                mctx1a_wt, mctx1b_wt, mctx1_b, mctx2_wt, mctx2_b,
                ro_w, out_ref,
                distw_r, fcm_r, dirx_r, diry_r, dirz_r,
                s_r, vx_r, vy_r, vz_r,
                sn_r, vxn_r, vyn_r, vzn_r, c_r):
    f32 = jnp.float32

    ii = jax.lax.broadcasted_iota(jnp.int32, (_N, _N), 0)
    jj = jax.lax.broadcasted_iota(jnp.int32, (_N, _N), 1)
    nself = (ii != jj).astype(f32)
    for b in range(_B):
        r0 = b * _N
        dx = posr[0:1, r0:r0 + _N] - posc[r0:r0 + _N, 0:1]
        dy = posr[1:2, r0:r0 + _N] - posc[r0:r0 + _N, 1:2]
        dz = posr[2:3, r0:r0 + _N] - posc[r0:r0 + _N, 2:3]
        dist = jnp.sqrt(dx * dx + dy * dy + dz * dz)
        rinv = 1.0 / (dist + 1e-8)
        fcut = 0.5 * (1.0 + jnp.cos(jnp.pi * (dist * (1.0 / _CUTOFF)))) \
            * (dist < _CUTOFF).astype(f32)
        fcm_r[r0:r0 + _N, :] = fcut * nself * _INV_SQRT_NB
        distw_r[r0:r0 + _N, :] = dist * (1.0 / _RBF_W)
        dirx_r[r0:r0 + _N, :] = dx * rinv
        diry_r[r0:r0 + _N, :] = dy * rinv
        dirz_r[r0:r0 + _N, :] = dz * rinv
    kf = jax.lax.broadcasted_iota(jnp.int32, (_NRBF, _RC, _N), 0).astype(f32)

    type_emb = _mm(onehot[...], atype_emb[...])
    t_emb = _mm(temb[...], tproj_wt[...]) + tproj_b[...]
    for b in range(_B):
        s_r[b * _N:(b + 1) * _N, :] = atom_emb[...] + type_emb + t_emb[b:b + 1, :]
    zero = jnp.zeros((_B * _N, _H), f32)
    vx_r[...] = zero
    vy_r[...] = zero
    vz_r[...] = zero

    for l in range(_NL):
        s_val = s_r[...]
        s_n = _ln(s_val, iln_g[l], iln_b[l])
        ctxh = _silu(_mm(s_n, ictx1_wt[l]) + ictx1_b[l])
        c_r[...] = _mm(ctxh, ictx2_wt[l]) + ictx2_b[l]
        f1w, f1b = filt1_wt[l], filt1_b[l]
        f2w, f2b = filt2_wt[l], filt2_b[l]

        def chunk(c, carry, f1w=f1w, f1b=f1b, f2w=f2w, f2b=f2b):
            r0 = c * _RC
            b0 = (r0 // _N) * _N
            dw = distw_r[pl.ds(r0, _RC), :]
            arg = dw[None, :, :] - kf
            rbf = jnp.exp(-0.5 * arg * arg)
            rbf2 = rbf.reshape(_NRBF, _RC * _N)
            h = jax.lax.dot_general(rbf2, f1w, (((0,), (0,)), ((), ())),
                                    preferred_element_type=f32)
            h = _silu(h + f1b)
            wf = _mm(h, f2w) + f2b
            wf3 = wf.reshape(_RC, _N, 3 * _H)
            fc = fcm_r[pl.ds(r0, _RC), :]
            Cb = c_r[pl.ds(b0, _N), :]
            msg = wf3 * fc[:, :, None] * Cb[None, :, :]
            ds = jnp.sum(msg[:, :, :_H], axis=1)
            dvs = msg[:, :, _H:2 * _H]
            dvv = msg[:, :, 2 * _H:]
            dxs = dirx_r[pl.ds(r0, _RC), :]
            dys = diry_r[pl.ds(r0, _RC), :]
            dzs = dirz_r[pl.ds(r0, _RC), :]
            vxb = vx_r[pl.ds(b0, _N), :]
            vyb = vy_r[pl.ds(b0, _N), :]
            vzb = vz_r[pl.ds(b0, _N), :]
            dvx = jnp.sum(dvs * dxs[:, :, None] + dvv * vxb[None, :, :], axis=1)
            dvy = jnp.sum(dvs * dys[:, :, None] + dvv * vyb[None, :, :], axis=1)
            dvz = jnp.sum(dvs * dzs[:, :, None] + dvv * vzb[None, :, :], axis=1)
            sn_r[pl.ds(r0, _RC), :] = s_r[pl.ds(r0, _RC), :] + ds
            vxn_r[pl.ds(r0, _RC), :] = vx_r[pl.ds(r0, _RC), :] + dvx
            vyn_r[pl.ds(r0, _RC), :] = vy_r[pl.ds(r0, _RC), :] + dvy
            vzn_r[pl.ds(r0, _RC), :] = vz_r[pl.ds(r0, _RC), :] + dvz
            return carry

        jax.lax.fori_loop(0, _NCHUNK, chunk, 0)

        s_val = sn_r[...]
        vx_val, vy_val, vz_val = vxn_r[...], vyn_r[...], vzn_r[...]
        uw, vw = u_wt[l], v_wt[l]
        Uvx, Uvy, Uvz = _mm(vx_val, uw), _mm(vy_val, uw), _mm(vz_val, uw)
        Vvx, Vvy, Vvz = _mm(vx_val, vw), _mm(vy_val, vw), _mm(vz_val, vw)
        vvn = jnp.sqrt(Vvx * Vvx + Vvy * Vvy + Vvz * Vvz + 1e-8)
        s_n2 = _ln(s_val, mln_g[l], mln_b[l])
        hid = _silu(_mm(s_n2, mctx1a_wt[l]) + _mm(vvn, mctx1b_wt[l]) + mctx1_b[l])
        ctx = _mm(hid, mctx2_wt[l]) + mctx2_b[l]
        a_ss = ctx[:, :_H]
        a_sv = ctx[:, _H:2 * _H]
        a_vv = ctx[:, 2 * _H:]
        dot_uv = Uvx * Vvx + Uvy * Vvy + Uvz * Vvz
        s_r[...] = s_val + a_ss + a_sv * dot_uv
        vx_r[...] = vx_val + a_vv * Uvx
        vy_r[...] = vy_val + a_vv * Uvy
        vz_r[...] = vz_val + a_vv * Uvz

    rw = ro_w[...]
    dn = (((1,), (1,)), ((), ()))
    velx = jax.lax.dot_general(vx_r[...], rw, dn, preferred_element_type=f32)
    vely = jax.lax.dot_general(vy_r[...], rw, dn, preferred_element_type=f32)
    velz = jax.lax.dot_general(vz_r[...], rw, dn, preferred_element_type=f32)
    out_ref[...] = jnp.concatenate([velx, vely, velz], axis=1)


def kernel(positions, t, atom_type_ids, params):
    pos = positions.reshape(_B * _N, 3).astype(jnp.float32)
    posr = pos.T
    half = _H // 2
    freqs = jnp.exp(-math.log(10000.0)
                    * jnp.arange(half, dtype=jnp.float32) / half)
    a = t[:, None] * freqs[None, :]
    temb = jnp.concatenate([jnp.sin(a), jnp.cos(a)], axis=-1)
    onehot = jax.nn.one_hot(atom_type_ids, 4, dtype=jnp.float32)

    ips = params["interactions"]
    mps = params["mixings"]

    def stk(xs):
        return jnp.stack(xs, axis=0)

    args = [
        pos, posr, temb, onehot,
        params["atom_embedding"],
        params["atom_type_embed"],
        params["time_proj"]["W"].T,
        params["time_proj"]["b"][None, :],
        stk([p["ln_g"][None, :] for p in ips]),
        stk([p["ln_b"][None, :] for p in ips]),
        stk([p["ctx1"]["W"].T for p in ips]),
        stk([p["ctx1"]["b"][None, :] for p in ips]),
        stk([p["ctx2"]["W"].T for p in ips]),
        stk([p["ctx2"]["b"][None, :] for p in ips]),
        stk([p["filt1"]["W"].T for p in ips]),
        stk([p["filt1"]["b"][None, :] for p in ips]),
        stk([p["filt2"]["W"].T for p in ips]),
        stk([p["filt2"]["b"][None, :] for p in ips]),
        stk([p["ln_g"][None, :] for p in mps]),
        stk([p["ln_b"][None, :] for p in mps]),
        stk([p["U"]["W"].T for p in mps]),
        stk([p["V"]["W"].T for p in mps]),
        stk([p["ctx1"]["W"].T[:_H] for p in mps]),
        stk([p["ctx1"]["W"].T[_H:] for p in mps]),
        stk([p["ctx1"]["b"][None, :] for p in mps]),
        stk([p["ctx2"]["W"].T for p in mps]),
        stk([p["ctx2"]["b"][None, :] for p in mps]),
        params["readout"]["W"],
    ]
    BN = _B * _N
    scratch = [
        pltpu.VMEM((BN, _N), jnp.float32),
        pltpu.VMEM((BN, _N), jnp.float32),
        pltpu.VMEM((BN, _N), jnp.float32),
        pltpu.VMEM((BN, _N), jnp.float32),
        pltpu.VMEM((BN, _N), jnp.float32),
        pltpu.VMEM((BN, _H), jnp.float32),
        pltpu.VMEM((BN, _H), jnp.float32),
        pltpu.VMEM((BN, _H), jnp.float32),
        pltpu.VMEM((BN, _H), jnp.float32),
        pltpu.VMEM((BN, _H), jnp.float32),
        pltpu.VMEM((BN, _H), jnp.float32),
        pltpu.VMEM((BN, _H), jnp.float32),
        pltpu.VMEM((BN, _H), jnp.float32),
        pltpu.VMEM((BN, 3 * _H), jnp.float32),
    ]
    out = pl.pallas_call(
        _painn_body,
        out_shape=jax.ShapeDtypeStruct((BN, 3), jnp.float32),
        scratch_shapes=scratch,
    )(*args)
    return out.reshape(_B, _N, 3)
